# baseline (speedup 1.0000x reference)
import functools

import jax
import jax.numpy as jnp
from jax import lax
from jax.experimental import pallas as pl
from jax.experimental.pallas import tpu as pltpu
from jax.experimental.pallas import tpu_sc as plsc

N = 10000
E = 320000
D = 128

NC = 2
NS = 16
NW = NC * NS

K = 96
EPT = E // NW
NCHUNK = 106
EPAD = NCHUNK * K - EPT
NP = N + 16
RPT = N // NS
ZPT = NP // NS

_SC_PARAMS = pltpu.CompilerParams(use_tc_tiling_on_sc=False)


def _sc_mesh():
    return plsc.VectorSubcoreMesh(
        core_axis_name="c", subcore_axis_name="s",
        num_cores=NC, num_subcores=NS)


def _cnt_call(dst3d, zc):
    out_type = (jax.ShapeDtypeStruct((NC, N, 16), jnp.float32),)
    scratch = (
        pltpu.VMEM((NCHUNK, K), jnp.int32),
        pltpu.VMEM((K, 16), jnp.float32),
        pltpu.VMEM_SHARED((NP, 16), jnp.float32),
        pltpu.SemaphoreType.DMA,
    )

    def body(dst_r, zc_r, cnto, idxd, ones_r, accc, sem_c):
        c = lax.axis_index("c")
        s = lax.axis_index("s")
        w = c * NS + s
        r0 = s * RPT
        z0 = s * ZPT
        pltpu.sync_copy(zc_r.at[pl.ds(z0, ZPT)], accc.at[pl.ds(z0, ZPT)])
        pltpu.sync_copy(dst_r.at[w], idxd)
        for i in range(K):
            ones_r[i] = jnp.ones((16,), jnp.float32)
        plsc.subcore_barrier()

        def step(ci, carry):
            pltpu.async_copy(ones_r, accc.at[idxd.at[ci]], sem_c, add=True)
            return carry

        lax.fori_loop(0, NCHUNK, step, 0)

        def drain(ci, carry):
            pltpu.make_async_copy(ones_r, accc.at[idxd.at[0]], sem_c).wait()
            return carry

        lax.fori_loop(0, NCHUNK, drain, 0)
        plsc.subcore_barrier()
        pltpu.sync_copy(accc.at[pl.ds(r0, RPT)], cnto.at[c, pl.ds(r0, RPT)])

    run = pl.kernel(body, out_type=out_type, mesh=_sc_mesh(),
                    scratch_types=scratch, compiler_params=_SC_PARAMS)
    return run(dst3d, zc)[0]


def _agg_call(feat, src3d, dst3d, zf, nbuf=2, look=1):
    assert 1 <= look <= nbuf - 1 and NCHUNK % nbuf == 0
    out_type = (jax.ShapeDtypeStruct((NC, N, D), jnp.float32),)
    scratch = [
        pltpu.VMEM((NCHUNK, K), jnp.int32),
        pltpu.VMEM((NCHUNK, K), jnp.int32),
        pltpu.VMEM_SHARED((NP, D), jnp.float32),
    ]
    scratch += [pltpu.VMEM((K, D), jnp.float32) for _ in range(nbuf)]
    scratch += [pltpu.SemaphoreType.DMA for _ in range(2 * nbuf)]

    def body(*refs):
        (feat_r, src_r, dst_r, zf_r, aggo, idxs, idxd, accf) = refs[:8]
        rows = refs[8:8 + nbuf]
        sem_g = refs[8 + nbuf:8 + 2 * nbuf]
        sem_s = refs[8 + 2 * nbuf:8 + 3 * nbuf]

        c = lax.axis_index("c")
        s = lax.axis_index("s")
        w = c * NS + s
        r0 = s * RPT
        z0 = s * ZPT

        def gather(ci, b):
            pltpu.async_copy(feat_r.at[idxs.at[ci]], rows[b], sem_g[b])

        def wait_gather(ci, b):
            pltpu.make_async_copy(
                feat_r.at[idxs.at[ci]], rows[b], sem_g[b]).wait()

        def scatter(ci, b):
            pltpu.async_copy(rows[b], accf.at[idxd.at[ci]], sem_s[b],
                             add=True)

        def wait_scatter(ci, b):
            pltpu.make_async_copy(rows[b], accf.at[idxd.at[ci]],
                                  sem_s[b]).wait()

        pltpu.sync_copy(zf_r.at[pl.ds(z0, ZPT)], accf.at[pl.ds(z0, ZPT)])
        pltpu.sync_copy(src_r.at[w], idxs)
        pltpu.sync_copy(dst_r.at[w], idxd)
        plsc.subcore_barrier()

        for ci in range(look):
            gather(ci, ci)

        def group(g, carry):
            for b in range(nbuf):
                ci = g * nbuf + b
                gi = ci + look
                bg = (b + look) % nbuf

                @pl.when(jnp.logical_and(gi >= nbuf, gi < NCHUNK))
                def _():
                    wait_scatter(gi - nbuf, bg)
                    gather(gi, bg)

                @pl.when(jnp.logical_and(gi < nbuf, gi < NCHUNK))
                def _():
                    gather(gi, bg)

                wait_gather(ci, b)
                scatter(ci, b)
            return carry

        lax.fori_loop(0, NCHUNK // nbuf, group, 0)
        for b in range(nbuf):
            wait_scatter(NCHUNK - nbuf + b, b)
        plsc.subcore_barrier()
        pltpu.sync_copy(accf.at[pl.ds(r0, RPT)], aggo.at[c, pl.ds(r0, RPT)])

    run = pl.kernel(body, out_type=out_type, mesh=_sc_mesh(),
                    scratch_types=tuple(scratch), compiler_params=_SC_PARAMS)
    return run(feat, src3d, dst3d, zf)[0]


def _dense_call(aggp, cntp, xin, wl_t, wr_t, b2d, apply_relu):
    R = 1000
    grid = (N // R,)

    def body(aggp_ref, cntp_ref, x_ref, wl_ref, wr_ref, b_ref, o_ref):
        agg = aggp_ref[0] + aggp_ref[1]
        cnt = cntp_ref[0] + cntp_ref[1]
        cdiv = jnp.maximum(cnt[:, :1], 1.0)
        t = (jnp.dot(agg, wl_ref[...], preferred_element_type=jnp.float32)
             / cdiv
             + jnp.dot(x_ref[...], wr_ref[...],
                       preferred_element_type=jnp.float32)
             + b_ref[...])
        nrm = jnp.sqrt(jnp.sum(t * t, axis=1, keepdims=True))
        t = t / jnp.maximum(nrm, 1e-12)
        if apply_relu:
            t = jnp.maximum(t, 0.0)
        o_ref[...] = t

    return pl.pallas_call(
        body,
        grid=grid,
        in_specs=[
            pl.BlockSpec((NC, R, D), lambda i: (0, i, 0)),
            pl.BlockSpec((NC, R, 16), lambda i: (0, i, 0)),
            pl.BlockSpec((R, D), lambda i: (i, 0)),
            pl.BlockSpec((D, D), lambda i: (0, 0)),
            pl.BlockSpec((D, D), lambda i: (0, 0)),
            pl.BlockSpec((1, D), lambda i: (0, 0)),
        ],
        out_specs=pl.BlockSpec((R, D), lambda i: (i, 0)),
        out_shape=jax.ShapeDtypeStruct((N, D), jnp.float32),
    )(aggp, cntp, xin, wl_t, wr_t, b2d)


def kernel(x, edge_index, W1_l, W1_r, b1, W2_l, W2_r, b2):
    src = edge_index[0].astype(jnp.int32).reshape(NW, EPT)
    dst = edge_index[1].astype(jnp.int32).reshape(NW, EPT)
    src = jnp.concatenate(
        [src, jnp.zeros((NW, EPAD), jnp.int32)], axis=1
    ).reshape(NW, NCHUNK, K)
    dst = jnp.concatenate(
        [dst, jnp.full((NW, EPAD), N, jnp.int32)], axis=1
    ).reshape(NW, NCHUNK, K)
    zf = jnp.zeros((NP, D), jnp.float32)
    zc = jnp.zeros((NP, 16), jnp.float32)

    cntp = _cnt_call(dst, zc)
    aggp1 = _agg_call(x, src, dst, zf)
    h = _dense_call(aggp1, cntp, x, W1_l.T, W1_r.T,
                    b1.reshape(1, D), apply_relu=True)
    aggp2 = _agg_call(h, src, dst, zf)
    out = _dense_call(aggp2, cntp, h, W2_l.T, W2_r.T,
                      b2.reshape(1, D), apply_relu=False)
    return out


# speedup vs baseline: 1.5730x; 1.5730x over previous
import functools

import jax
import jax.numpy as jnp
from jax import lax
from jax.experimental import pallas as pl
from jax.experimental.pallas import tpu as pltpu
from jax.experimental.pallas import tpu_sc as plsc

N = 10000
E = 320000
D = 128

NC = 2
NS = 16
NW = NC * NS

K = 48
EPT = E // NW
NCHUNK = 210
EPAD = NCHUNK * K - EPT
NP = N + 16
RPT = N // NS
ZPT = NP // NS

_SC_PARAMS = pltpu.CompilerParams(use_tc_tiling_on_sc=False)


def _sc_mesh():
    return plsc.VectorSubcoreMesh(
        core_axis_name="c", subcore_axis_name="s",
        num_cores=NC, num_subcores=NS)


def _cnt_call(dst3d, zc):
    out_type = (jax.ShapeDtypeStruct((NC, N, 16), jnp.float32),)
    scratch = (
        pltpu.VMEM((NCHUNK, K), jnp.int32),
        pltpu.VMEM((K, 16), jnp.float32),
        pltpu.VMEM_SHARED((NP, 16), jnp.float32),
        pltpu.SemaphoreType.DMA,
    )

    def body(dst_r, zc_r, cnto, idxd, ones_r, accc, sem_c):
        c = lax.axis_index("c")
        s = lax.axis_index("s")
        w = c * NS + s
        r0 = s * RPT
        z0 = s * ZPT
        pltpu.sync_copy(zc_r.at[pl.ds(z0, ZPT)], accc.at[pl.ds(z0, ZPT)])
        pltpu.sync_copy(dst_r.at[w], idxd)
        for i in range(K):
            ones_r[i] = jnp.ones((16,), jnp.float32)
        plsc.subcore_barrier()

        def step(ci, carry):
            pltpu.async_copy(ones_r, accc.at[idxd.at[ci]], sem_c, add=True)
            return carry

        lax.fori_loop(0, NCHUNK, step, 0)

        def drain(ci, carry):
            pltpu.make_async_copy(ones_r, accc.at[idxd.at[0]], sem_c).wait()
            return carry

        lax.fori_loop(0, NCHUNK, drain, 0)
        plsc.subcore_barrier()
        pltpu.sync_copy(accc.at[pl.ds(r0, RPT)], cnto.at[c, pl.ds(r0, RPT)])

    run = pl.kernel(body, out_type=out_type, mesh=_sc_mesh(),
                    scratch_types=scratch, compiler_params=_SC_PARAMS)
    return run(dst3d, zc)[0]


def _agg_call(feat, src3d, dst3d, zf, nbuf=5, look=3):
    assert 1 <= look <= nbuf - 1 and NCHUNK % nbuf == 0
    out_type = (jax.ShapeDtypeStruct((NC, N, D), jnp.float32),)
    scratch = [
        pltpu.VMEM((NCHUNK, K), jnp.int32),
        pltpu.VMEM((NCHUNK, K), jnp.int32),
        pltpu.VMEM_SHARED((NP, D), jnp.float32),
    ]
    scratch += [pltpu.VMEM((K, D), jnp.float32) for _ in range(nbuf)]
    scratch += [pltpu.SemaphoreType.DMA for _ in range(2 * nbuf)]

    def body(*refs):
        (feat_r, src_r, dst_r, zf_r, aggo, idxs, idxd, accf) = refs[:8]
        rows = refs[8:8 + nbuf]
        sem_g = refs[8 + nbuf:8 + 2 * nbuf]
        sem_s = refs[8 + 2 * nbuf:8 + 3 * nbuf]

        c = lax.axis_index("c")
        s = lax.axis_index("s")
        w = c * NS + s
        r0 = s * RPT
        z0 = s * ZPT

        def gather(ci, b):
            pltpu.async_copy(feat_r.at[idxs.at[ci]], rows[b], sem_g[b])

        def wait_gather(ci, b):
            pltpu.make_async_copy(
                feat_r.at[idxs.at[ci]], rows[b], sem_g[b]).wait()

        def scatter(ci, b):
            pltpu.async_copy(rows[b], accf.at[idxd.at[ci]], sem_s[b],
                             add=True)

        def wait_scatter(ci, b):
            pltpu.make_async_copy(rows[b], accf.at[idxd.at[ci]],
                                  sem_s[b]).wait()

        pltpu.sync_copy(zf_r.at[pl.ds(z0, ZPT)], accf.at[pl.ds(z0, ZPT)])
        pltpu.sync_copy(src_r.at[w], idxs)
        pltpu.sync_copy(dst_r.at[w], idxd)
        plsc.subcore_barrier()

        for ci in range(look):
            gather(ci, ci)

        def group(g, carry):
            for b in range(nbuf):
                ci = g * nbuf + b
                gi = ci + look
                bg = (b + look) % nbuf

                @pl.when(jnp.logical_and(gi >= nbuf, gi < NCHUNK))
                def _():
                    wait_scatter(gi - nbuf, bg)
                    gather(gi, bg)

                @pl.when(jnp.logical_and(gi < nbuf, gi < NCHUNK))
                def _():
                    gather(gi, bg)

                wait_gather(ci, b)
                scatter(ci, b)
            return carry

        lax.fori_loop(0, NCHUNK // nbuf, group, 0)
        for b in range(nbuf):
            wait_scatter(NCHUNK - nbuf + b, b)
        plsc.subcore_barrier()
        pltpu.sync_copy(accf.at[pl.ds(r0, RPT)], aggo.at[c, pl.ds(r0, RPT)])

    run = pl.kernel(body, out_type=out_type, mesh=_sc_mesh(),
                    scratch_types=tuple(scratch), compiler_params=_SC_PARAMS)
    return run(feat, src3d, dst3d, zf)[0]


def _dense_call(aggp, cntp, xin, wl_t, wr_t, b2d, apply_relu):
    R = 1000
    grid = (N // R,)

    def body(aggp_ref, cntp_ref, x_ref, wl_ref, wr_ref, b_ref, o_ref):
        agg = aggp_ref[0] + aggp_ref[1]
        cnt = cntp_ref[0] + cntp_ref[1]
        cdiv = jnp.maximum(cnt[:, :1], 1.0)
        t = (jnp.dot(agg, wl_ref[...], preferred_element_type=jnp.float32)
             / cdiv
             + jnp.dot(x_ref[...], wr_ref[...],
                       preferred_element_type=jnp.float32)
             + b_ref[...])
        nrm = jnp.sqrt(jnp.sum(t * t, axis=1, keepdims=True))
        t = t / jnp.maximum(nrm, 1e-12)
        if apply_relu:
            t = jnp.maximum(t, 0.0)
        o_ref[...] = t

    return pl.pallas_call(
        body,
        grid=grid,
        in_specs=[
            pl.BlockSpec((NC, R, D), lambda i: (0, i, 0)),
            pl.BlockSpec((NC, R, 16), lambda i: (0, i, 0)),
            pl.BlockSpec((R, D), lambda i: (i, 0)),
            pl.BlockSpec((D, D), lambda i: (0, 0)),
            pl.BlockSpec((D, D), lambda i: (0, 0)),
            pl.BlockSpec((1, D), lambda i: (0, 0)),
        ],
        out_specs=pl.BlockSpec((R, D), lambda i: (i, 0)),
        out_shape=jax.ShapeDtypeStruct((N, D), jnp.float32),
    )(aggp, cntp, xin, wl_t, wr_t, b2d)


def kernel(x, edge_index, W1_l, W1_r, b1, W2_l, W2_r, b2):
    src = edge_index[0].astype(jnp.int32).reshape(NW, EPT)
    dst = edge_index[1].astype(jnp.int32).reshape(NW, EPT)
    src = jnp.concatenate(
        [src, jnp.zeros((NW, EPAD), jnp.int32)], axis=1
    ).reshape(NW, NCHUNK, K)
    dst = jnp.concatenate(
        [dst, jnp.full((NW, EPAD), N, jnp.int32)], axis=1
    ).reshape(NW, NCHUNK, K)
    zf = jnp.zeros((NP, D), jnp.float32)
    zc = jnp.zeros((NP, 16), jnp.float32)

    cntp = _cnt_call(dst, zc)
    aggp1 = _agg_call(x, src, dst, zf)
    h = _dense_call(aggp1, cntp, x, W1_l.T, W1_r.T,
                    b1.reshape(1, D), apply_relu=True)
    aggp2 = _agg_call(h, src, dst, zf)
    out = _dense_call(aggp2, cntp, h, W2_l.T, W2_r.T,
                      b2.reshape(1, D), apply_relu=False)
    return out


# speedup vs baseline: 2.6512x; 1.6855x over previous
import functools

import jax
import jax.numpy as jnp
from jax import lax
from jax.experimental import pallas as pl
from jax.experimental.pallas import tpu as pltpu
from jax.experimental.pallas import tpu_sc as plsc

N = 10000
E = 320000
D = 128

NC = 2
NS = 16
NW = NC * NS

K = 40
EPT = E // NW
NCHUNK = 250
EPAD = NCHUNK * K - EPT
NP = N + 16
RPT = N // NS
ZPT = NP // NS

_SC_PARAMS = pltpu.CompilerParams(use_tc_tiling_on_sc=False)


def _sc_mesh():
    return plsc.VectorSubcoreMesh(
        core_axis_name="c", subcore_axis_name="s",
        num_cores=NC, num_subcores=NS)


def _cnt_call(dst3d, zc):
    out_type = (jax.ShapeDtypeStruct((NC, N, 16), jnp.float32),)
    scratch = (
        pltpu.VMEM((NCHUNK, K), jnp.int32),
        pltpu.VMEM((K, 16), jnp.float32),
        pltpu.VMEM_SHARED((NP, 16), jnp.float32),
        pltpu.SemaphoreType.DMA,
    )

    def body(dst_r, zc_r, cnto, idxd, ones_r, accc, sem_c):
        c = lax.axis_index("c")
        s = lax.axis_index("s")
        w = c * NS + s
        r0 = s * RPT
        z0 = s * ZPT
        pltpu.sync_copy(zc_r.at[pl.ds(z0, ZPT)], accc.at[pl.ds(z0, ZPT)])
        pltpu.sync_copy(dst_r.at[w], idxd)
        for i in range(K):
            ones_r[i] = jnp.ones((16,), jnp.float32)
        plsc.subcore_barrier()

        def step(ci, carry):
            pltpu.async_copy(ones_r, accc.at[idxd.at[ci]], sem_c, add=True)
            return carry

        lax.fori_loop(0, NCHUNK, step, 0)

        def drain(ci, carry):
            pltpu.make_async_copy(ones_r, accc.at[idxd.at[0]], sem_c).wait()
            return carry

        lax.fori_loop(0, NCHUNK, drain, 0)
        plsc.subcore_barrier()
        pltpu.sync_copy(accc.at[pl.ds(r0, RPT)], cnto.at[c, pl.ds(r0, RPT)])

    run = pl.kernel(body, out_type=out_type, mesh=_sc_mesh(),
                    scratch_types=scratch, compiler_params=_SC_PARAMS)
    return run(dst3d, zc)[0]


def _agg_call(feat, src3d, dst3d, zf, nbuf=5, look=3):
    assert 1 <= look <= nbuf - 1 and NCHUNK % nbuf == 0
    out_type = (jax.ShapeDtypeStruct((NC, N, D), jnp.float32),)
    scratch = [
        pltpu.VMEM((NCHUNK, K), jnp.int32),
        pltpu.VMEM((NCHUNK, K), jnp.int32),
        pltpu.VMEM_SHARED((NP, D), jnp.float32),
    ]
    scratch += [pltpu.VMEM((K, D), jnp.float32) for _ in range(nbuf)]
    scratch += [pltpu.SemaphoreType.DMA for _ in range(2 * nbuf)]

    def body(*refs):
        (feat_r, src_r, dst_r, zf_r, aggo, idxs, idxd, accf) = refs[:8]
        rows = refs[8:8 + nbuf]
        sem_g = refs[8 + nbuf:8 + 2 * nbuf]
        sem_s = refs[8 + 2 * nbuf:8 + 3 * nbuf]

        c = lax.axis_index("c")
        s = lax.axis_index("s")
        w = c * NS + s
        r0 = s * RPT
        z0 = s * ZPT

        def gather(ci, b):
            pltpu.async_copy(feat_r.at[idxs.at[ci]], rows[b], sem_g[b])

        def wait_gather(ci, b):
            pltpu.make_async_copy(
                feat_r.at[idxs.at[ci]], rows[b], sem_g[b]).wait()

        def scatter(ci, b):
            pltpu.async_copy(rows[b], accf.at[idxd.at[ci]], sem_s[b],
                             add=True)

        def wait_scatter(ci, b):
            pltpu.make_async_copy(rows[b], accf.at[idxd.at[ci]],
                                  sem_s[b]).wait()

        pltpu.sync_copy(zf_r.at[pl.ds(z0, ZPT)], accf.at[pl.ds(z0, ZPT)])
        pltpu.sync_copy(src_r.at[w], idxs)
        pltpu.sync_copy(dst_r.at[w], idxd)
        plsc.subcore_barrier()

        for ci in range(look):
            gather(ci, ci)

        def group(g, carry):
            for b in range(nbuf):
                ci = g * nbuf + b
                gi = ci + look
                bg = (b + look) % nbuf

                @pl.when(jnp.logical_and(gi >= nbuf, gi < NCHUNK))
                def _():
                    wait_scatter(gi - nbuf, bg)
                    gather(gi, bg)

                @pl.when(jnp.logical_and(gi < nbuf, gi < NCHUNK))
                def _():
                    gather(gi, bg)

                wait_gather(ci, b)
                scatter(ci, b)
            return carry

        lax.fori_loop(0, NCHUNK // nbuf, group, 0)
        for b in range(nbuf):
            wait_scatter(NCHUNK - nbuf + b, b)
        plsc.subcore_barrier()
        pltpu.sync_copy(accf.at[pl.ds(r0, RPT)], aggo.at[c, pl.ds(r0, RPT)])

    run = pl.kernel(body, out_type=out_type, mesh=_sc_mesh(),
                    scratch_types=tuple(scratch), compiler_params=_SC_PARAMS)
    return run(feat, src3d, dst3d, zf)[0]


def _dense_call(aggp, cntp, xin, wl_t, wr_t, b2d, apply_relu):
    R = 1000
    grid = (N // R,)

    def body(aggp_ref, cntp_ref, x_ref, wl_ref, wr_ref, b_ref, o_ref):
        agg = aggp_ref[0] + aggp_ref[1]
        cnt = cntp_ref[0] + cntp_ref[1]
        cdiv = jnp.maximum(cnt[:, :1], 1.0)
        t = (jnp.dot(agg, wl_ref[...], preferred_element_type=jnp.float32)
             / cdiv
             + jnp.dot(x_ref[...], wr_ref[...],
                       preferred_element_type=jnp.float32)
             + b_ref[...])
        nrm = jnp.sqrt(jnp.sum(t * t, axis=1, keepdims=True))
        t = t / jnp.maximum(nrm, 1e-12)
        if apply_relu:
            t = jnp.maximum(t, 0.0)
        o_ref[...] = t

    return pl.pallas_call(
        body,
        grid=grid,
        in_specs=[
            pl.BlockSpec((NC, R, D), lambda i: (0, i, 0)),
            pl.BlockSpec((NC, R, 16), lambda i: (0, i, 0)),
            pl.BlockSpec((R, D), lambda i: (i, 0)),
            pl.BlockSpec((D, D), lambda i: (0, 0)),
            pl.BlockSpec((D, D), lambda i: (0, 0)),
            pl.BlockSpec((1, D), lambda i: (0, 0)),
        ],
        out_specs=pl.BlockSpec((R, D), lambda i: (i, 0)),
        out_shape=jax.ShapeDtypeStruct((N, D), jnp.float32),
    )(aggp, cntp, xin, wl_t, wr_t, b2d)


def kernel(x, edge_index, W1_l, W1_r, b1, W2_l, W2_r, b2):
    src = edge_index[0].astype(jnp.int32).reshape(NW, EPT)
    dst = edge_index[1].astype(jnp.int32).reshape(NW, EPT)
    if EPAD:
        src = jnp.concatenate(
            [src, jnp.zeros((NW, EPAD), jnp.int32)], axis=1)
        dst = jnp.concatenate(
            [dst, jnp.full((NW, EPAD), N, jnp.int32)], axis=1)
    src = src.reshape(NW, NCHUNK, K)
    dst = dst.reshape(NW, NCHUNK, K)
    zf = jnp.zeros((NP, D), jnp.float32)
    zc = jnp.zeros((NP, 16), jnp.float32)

    cntp = _cnt_call(dst, zc)
    aggp1 = _agg_call(x, src, dst, zf)
    h = _dense_call(aggp1, cntp, x, W1_l.T, W1_r.T,
                    b1.reshape(1, D), apply_relu=True)
    aggp2 = _agg_call(h, src, dst, zf)
    out = _dense_call(aggp2, cntp, h, W2_l.T, W2_r.T,
                      b2.reshape(1, D), apply_relu=False)
    return out
